# baseline (speedup 1.0000x reference)
import numpy as np
import jax
import jax.numpy as jnp
from jax.experimental import pallas as pl
from jax.experimental.pallas import tpu as pltpu

_H_DISC = 4
_W_DISC = 4
_DISC = _H_DISC * _W_DISC


def _interp_matrix(n_in: int, n_out: int) -> np.ndarray:
    a = np.zeros((n_out, n_in), np.float64)
    s = n_in / n_out
    for i in range(n_out):
        x = (i + 0.5) * s - 0.5
        lo = int(np.floor(x))
        t = x - lo
        for idx, w in ((lo, 1.0 - t), (lo + 1, t)):
            a[i, min(max(idx, 0), n_in - 1)] += w
    return a.astype(np.float32)


def _upsample_kernel(emb_ref, b_ref, f_ref, o_ref):
    cont = jnp.dot(emb_ref[...], b_ref[...],
                   preferred_element_type=jnp.float32)
    o_ref[...] = f_ref[...] + cont[None, :, :]


def kernel(f, emb_w):
    batch, seq, ch, fh, fw = f.shape
    hw = fh * fw

    a_h = _interp_matrix(_H_DISC, fh)
    a_w = _interp_matrix(_W_DISC, fw)
    b_mat = jnp.asarray(np.kron(a_h, a_w).T)

    f3 = f.reshape(batch, seq * ch, hw)
    emb3 = emb_w.reshape(emb_w.shape[0] * ch, _DISC)

    seq_block = 4
    rows = seq_block * ch
    grid = (seq // seq_block,)

    out = pl.pallas_call(
        _upsample_kernel,
        grid=grid,
        in_specs=[
            pl.BlockSpec((rows, _DISC), lambda i: (i, 0)),
            pl.BlockSpec((_DISC, hw), lambda i: (0, 0)),
            pl.BlockSpec((batch, rows, hw), lambda i: (0, i, 0)),
        ],
        out_specs=pl.BlockSpec((batch, rows, hw), lambda i: (0, i, 0)),
        out_shape=jax.ShapeDtypeStruct((batch, seq * ch, hw), jnp.float32),
        compiler_params=pltpu.CompilerParams(
            dimension_semantics=("arbitrary",),
        ),
    )(emb3, b_mat, f3)
    return out.reshape(f.shape)
